# baseline (speedup 1.0000x reference)
import functools

import jax
import jax.numpy as jnp
from jax import lax
from jax.experimental import pallas as pl
from jax.experimental.pallas import tpu as pltpu
from jax.experimental.pallas import tpu_sc as plsc

_NUM_CORES = 2
_NUM_SUBCORES = 16
_NUM_WORKERS = _NUM_CORES * _NUM_SUBCORES
_LANES = 16


def _sc_gather_mean(ctx_flat, emb_table, batch, ctx_len, embed):
    rows_per_worker = batch // _NUM_WORKERS
    idx_per_worker = rows_per_worker * ctx_len
    rows_per_chunk = 4
    idx_per_chunk = rows_per_chunk * ctx_len
    num_chunks = rows_per_worker // rows_per_chunk

    mesh = plsc.VectorSubcoreMesh(core_axis_name="c", subcore_axis_name="s")

    @functools.partial(
        pl.kernel,
        out_type=jax.ShapeDtypeStruct((batch, embed), jnp.float32),
        mesh=mesh,
        scratch_types=[
            pltpu.VMEM((idx_per_worker,), jnp.int32),
            pltpu.VMEM((idx_per_chunk, embed), jnp.float32),
            pltpu.VMEM((rows_per_chunk, embed), jnp.float32),
        ],
        compiler_params=pltpu.CompilerParams(use_tc_tiling_on_sc=False),
    )
    def gather_mean(table_hbm, idx_hbm, out_hbm, idx_v, rows_v, acc_v):
        wid = lax.axis_index("s") * _NUM_CORES + lax.axis_index("c")
        idx_base = wid * idx_per_worker
        row_base = wid * rows_per_worker
        pltpu.sync_copy(idx_hbm.at[pl.ds(idx_base, idx_per_worker)], idx_v)

        scale = jnp.float32(1.0 / ctx_len)

        @pl.loop(0, num_chunks)
        def _(c):
            pltpu.sync_copy(
                table_hbm.at[idx_v.at[pl.ds(c * idx_per_chunk, idx_per_chunk)]],
                rows_v,
            )
            for r in range(rows_per_chunk):
                for q in range(embed // _LANES):
                    sl = pl.ds(q * _LANES, _LANES)
                    acc = rows_v[r * ctx_len, sl]
                    for j in range(1, ctx_len):
                        acc = acc + rows_v[r * ctx_len + j, sl]
                    acc_v[r, sl] = acc * scale
            pltpu.sync_copy(
                acc_v, out_hbm.at[pl.ds(row_base + c * rows_per_chunk,
                                        rows_per_chunk)]
            )

    return gather_mean(emb_table, ctx_flat)


def _mm_body(means_ref, w_ref, b_ref, out_ref):
    m = means_ref[...].astype(jnp.bfloat16)
    w = w_ref[...].astype(jnp.bfloat16)
    acc = lax.dot_general(
        m, w, (((1,), (1,)), ((), ())), preferred_element_type=jnp.float32
    )
    out_ref[...] = acc + b_ref[...]


def _tc_matmul(means, W, b2d, batch, embed, vocab):
    tile_v = 2048
    grid = (vocab + tile_v - 1) // tile_v
    return pl.pallas_call(
        _mm_body,
        grid=(grid,),
        in_specs=[
            pl.BlockSpec((batch, embed), lambda i: (0, 0)),
            pl.BlockSpec((tile_v, embed), lambda i: (i, 0)),
            pl.BlockSpec((1, tile_v), lambda i: (0, i)),
        ],
        out_specs=pl.BlockSpec((batch, tile_v), lambda i: (0, i)),
        out_shape=jax.ShapeDtypeStruct((batch, vocab), jnp.float32),
        compiler_params=pltpu.CompilerParams(
            dimension_semantics=("parallel",),
        ),
    )(means, W, b2d)


def kernel(context, emb_table, W, b):
    batch, ctx_len = context.shape
    vocab, embed = emb_table.shape
    ctx_flat = context.reshape(-1).astype(jnp.int32)
    means = _sc_gather_mean(ctx_flat, emb_table, batch, ctx_len, embed)
    return _tc_matmul(means, W, b.reshape(1, vocab), batch, embed, vocab)


# speedup vs baseline: 1.0198x; 1.0198x over previous
import functools

import jax
import jax.numpy as jnp
from jax import lax
from jax.experimental import pallas as pl
from jax.experimental.pallas import tpu as pltpu
from jax.experimental.pallas import tpu_sc as plsc

_NUM_CORES = 2
_NUM_SUBCORES = 16
_NUM_WORKERS = _NUM_CORES * _NUM_SUBCORES
_LANES = 16
_ROW = 128


def _sc_gather_mean(ctx_flat, emb_pad, batch, ctx_len, embed):
    rows_per_worker = batch // _NUM_WORKERS
    idx_per_worker = rows_per_worker * ctx_len
    rows_per_chunk = 4
    idx_per_chunk = rows_per_chunk * ctx_len
    num_chunks = rows_per_worker // rows_per_chunk

    mesh = plsc.VectorSubcoreMesh(core_axis_name="c", subcore_axis_name="s")

    @functools.partial(
        pl.kernel,
        out_type=jax.ShapeDtypeStruct((batch, embed), jnp.float32),
        mesh=mesh,
        scratch_types=[
            pltpu.VMEM((idx_per_worker,), jnp.int32),
            pltpu.VMEM((idx_per_chunk, _ROW), jnp.float32),
            pltpu.VMEM((rows_per_worker, embed), jnp.float32),
        ],
    )
    def gather_mean(table_hbm, idx_hbm, out_hbm, idx_v, rows_v, acc_v):
        wid = lax.axis_index("s") * _NUM_CORES + lax.axis_index("c")
        idx_base = wid * idx_per_worker
        row_base = wid * rows_per_worker
        pltpu.sync_copy(idx_hbm.at[pl.ds(idx_base, idx_per_worker)], idx_v)

        scale = jnp.float32(1.0 / ctx_len)

        @pl.loop(0, num_chunks)
        def _(c):
            pltpu.sync_copy(
                table_hbm.at[idx_v.at[pl.ds(c * idx_per_chunk, idx_per_chunk)]],
                rows_v,
            )
            for r in range(rows_per_chunk):
                for q in range(embed // _LANES):
                    sl = pl.ds(q * _LANES, _LANES)
                    acc = rows_v[r * ctx_len, sl]
                    for j in range(1, ctx_len):
                        acc = acc + rows_v[r * ctx_len + j, sl]
                    acc_v[c * rows_per_chunk + r, sl] = acc * scale

        pltpu.sync_copy(acc_v, out_hbm.at[pl.ds(row_base, rows_per_worker)])

    return gather_mean(emb_pad, ctx_flat)


def _mm_body(means_ref, w_ref, b_ref, out_ref):
    m = means_ref[...].astype(jnp.bfloat16)
    w = w_ref[...].astype(jnp.bfloat16)
    acc = lax.dot_general(
        m, w, (((1,), (1,)), ((), ())), preferred_element_type=jnp.float32
    )
    out_ref[...] = acc + b_ref[...][None, :]


def _tc_matmul(means, W, b, batch, embed, vocab):
    tile_v = 2048
    grid = (vocab + tile_v - 1) // tile_v
    return pl.pallas_call(
        _mm_body,
        grid=(grid,),
        in_specs=[
            pl.BlockSpec((batch, embed), lambda i: (0, 0)),
            pl.BlockSpec((tile_v, embed), lambda i: (i, 0)),
            pl.BlockSpec((tile_v,), lambda i: (i,)),
        ],
        out_specs=pl.BlockSpec((batch, tile_v), lambda i: (0, i)),
        out_shape=jax.ShapeDtypeStruct((batch, vocab), jnp.float32),
        compiler_params=pltpu.CompilerParams(
            dimension_semantics=("parallel",),
        ),
    )(means, W, b)


def kernel(context, emb_table, W, b):
    batch, ctx_len = context.shape
    vocab, embed = emb_table.shape
    ctx_flat = context.reshape(-1).astype(jnp.int32)
    emb_pad = jnp.pad(emb_table, ((0, 0), (0, _ROW - embed)))
    means = _sc_gather_mean(ctx_flat, emb_pad, batch, ctx_len, embed)
    return _tc_matmul(means, W, b, batch, embed, vocab)


# speedup vs baseline: 2.7089x; 2.6565x over previous
import functools

import jax
import jax.numpy as jnp
from jax import lax
from jax.experimental import pallas as pl
from jax.experimental.pallas import tpu as pltpu
from jax.experimental.pallas import tpu_sc as plsc

_NUM_CORES = 2
_NUM_SUBCORES = 16
_NUM_WORKERS = _NUM_CORES * _NUM_SUBCORES
_LANES = 16


def _sc_gather_mean(ctx_flat, emb_table, batch, ctx_len, embed):
    rows_per_worker = batch // _NUM_WORKERS
    idx_per_worker = rows_per_worker * ctx_len
    rows_per_chunk = 4
    idx_per_chunk = rows_per_chunk * ctx_len
    num_chunks = rows_per_worker // rows_per_chunk

    mesh = plsc.VectorSubcoreMesh(core_axis_name="c", subcore_axis_name="s")

    @functools.partial(
        pl.kernel,
        out_type=jax.ShapeDtypeStruct((batch, embed), jnp.float32),
        mesh=mesh,
        scratch_types=[
            pltpu.VMEM((idx_per_worker,), jnp.int32),
            pltpu.VMEM((idx_per_chunk, embed), jnp.float32),
            pltpu.VMEM((rows_per_worker, embed), jnp.float32),
        ],
        compiler_params=pltpu.CompilerParams(use_tc_tiling_on_sc=False),
    )
    def gather_mean(table_hbm, idx_hbm, out_hbm, idx_v, rows_v, acc_v):
        wid = lax.axis_index("s") * _NUM_CORES + lax.axis_index("c")
        idx_base = wid * idx_per_worker
        row_base = wid * rows_per_worker
        pltpu.sync_copy(idx_hbm.at[pl.ds(idx_base, idx_per_worker)], idx_v)

        scale = jnp.float32(1.0 / ctx_len)

        @pl.loop(0, num_chunks)
        def _(c):
            pltpu.sync_copy(
                table_hbm.at[idx_v.at[pl.ds(c * idx_per_chunk, idx_per_chunk)]],
                rows_v,
            )
            for r in range(rows_per_chunk):
                for q in range(embed // _LANES):
                    sl = pl.ds(q * _LANES, _LANES)
                    acc = rows_v[r * ctx_len, sl]
                    for j in range(1, ctx_len):
                        acc = acc + rows_v[r * ctx_len + j, sl]
                    acc_v[c * rows_per_chunk + r, sl] = acc * scale

        pltpu.sync_copy(acc_v, out_hbm.at[pl.ds(row_base, rows_per_worker)])

    return gather_mean(emb_table, ctx_flat)


def _mm_body(means_ref, wt_ref, b_ref, out_ref):
    m = means_ref[...].astype(jnp.bfloat16)
    wt = wt_ref[...].astype(jnp.bfloat16)
    acc = lax.dot_general(
        wt, m, (((0,), (1,)), ((), ())), preferred_element_type=jnp.float32
    )
    out_ref[...] = acc + b_ref[...][:, None]


def _tc_matmul_t(means, Wt, b, batch, embed, vocab):
    tile_v = 2048
    grid = (vocab + tile_v - 1) // tile_v
    return pl.pallas_call(
        _mm_body,
        grid=(grid,),
        in_specs=[
            pl.BlockSpec((batch, embed), lambda i: (0, 0)),
            pl.BlockSpec((embed, tile_v), lambda i: (0, i)),
            pl.BlockSpec((tile_v,), lambda i: (i,)),
        ],
        out_specs=pl.BlockSpec((tile_v, batch), lambda i: (i, 0)),
        out_shape=jax.ShapeDtypeStruct((vocab, batch), jnp.float32),
        compiler_params=pltpu.CompilerParams(
            dimension_semantics=("parallel",),
        ),
    )(means, Wt, b)


def kernel(context, emb_table, W, b):
    batch, ctx_len = context.shape
    vocab, embed = emb_table.shape
    ctx_flat = context.reshape(-1).astype(jnp.int32)
    means = _sc_gather_mean(ctx_flat, emb_table, batch, ctx_len, embed)
    out_t = _tc_matmul_t(means, W.T, b, batch, embed, vocab)
    return out_t.T


# speedup vs baseline: 3.0400x; 1.1222x over previous
import functools

import jax
import jax.numpy as jnp
from jax import lax
from jax.experimental import pallas as pl
from jax.experimental.pallas import tpu as pltpu
from jax.experimental.pallas import tpu_sc as plsc

_NUM_CORES = 2
_NUM_SUBCORES = 16
_NUM_WORKERS = _NUM_CORES * _NUM_SUBCORES
_LANES = 16
_ROW = 128


def _t_body(embed, in_ref, out_ref):
    out_ref[:, :embed] = in_ref[...].T


def _transpose_pad(emb_t, vocab, embed):
    tile_v = 4096
    grid = (vocab + tile_v - 1) // tile_v
    return pl.pallas_call(
        functools.partial(_t_body, embed),
        grid=(grid,),
        in_specs=[pl.BlockSpec((embed, tile_v), lambda i: (0, i))],
        out_specs=pl.BlockSpec((tile_v, _ROW), lambda i: (i, 0)),
        out_shape=jax.ShapeDtypeStruct((vocab, _ROW), jnp.float32),
        compiler_params=pltpu.CompilerParams(
            dimension_semantics=("parallel",),
        ),
    )(emb_t)


def _sc_gather_mean(ctx_flat, table_pad, batch, ctx_len, embed):
    rows_per_worker = batch // _NUM_WORKERS
    idx_per_worker = rows_per_worker * ctx_len
    rows_per_chunk = 4
    idx_per_chunk = rows_per_chunk * ctx_len
    num_chunks = rows_per_worker // rows_per_chunk

    mesh = plsc.VectorSubcoreMesh(core_axis_name="c", subcore_axis_name="s")

    @functools.partial(
        pl.kernel,
        out_type=jax.ShapeDtypeStruct((batch, embed), jnp.float32),
        mesh=mesh,
        scratch_types=[
            pltpu.VMEM((idx_per_worker,), jnp.int32),
            pltpu.VMEM((idx_per_chunk, _ROW), jnp.float32),
            pltpu.VMEM((rows_per_worker, embed), jnp.float32),
        ],
    )
    def gather_mean(table_hbm, idx_hbm, out_hbm, idx_v, rows_v, acc_v):
        wid = lax.axis_index("s") * _NUM_CORES + lax.axis_index("c")
        idx_base = wid * idx_per_worker
        row_base = wid * rows_per_worker
        pltpu.sync_copy(idx_hbm.at[pl.ds(idx_base, idx_per_worker)], idx_v)

        scale = jnp.float32(1.0 / ctx_len)

        @pl.loop(0, num_chunks)
        def _(c):
            pltpu.sync_copy(
                table_hbm.at[idx_v.at[pl.ds(c * idx_per_chunk, idx_per_chunk)]],
                rows_v,
            )
            for r in range(rows_per_chunk):
                for q in range(embed // _LANES):
                    sl = pl.ds(q * _LANES, _LANES)
                    acc = rows_v[r * ctx_len, sl]
                    for j in range(1, ctx_len):
                        acc = acc + rows_v[r * ctx_len + j, sl]
                    acc_v[c * rows_per_chunk + r, sl] = acc * scale

        pltpu.sync_copy(acc_v, out_hbm.at[pl.ds(row_base, rows_per_worker)])

    return gather_mean(table_pad, ctx_flat)


def _mm_body(means_ref, wt_ref, b_ref, out_ref):
    m = means_ref[...].astype(jnp.bfloat16)
    wt = wt_ref[...].astype(jnp.bfloat16)
    acc = lax.dot_general(
        wt, m, (((0,), (1,)), ((), ())), preferred_element_type=jnp.float32
    )
    out_ref[...] = acc + b_ref[...][:, None]


def _tc_matmul_t(means, Wt, b, batch, embed, vocab):
    tile_v = 2048
    grid = (vocab + tile_v - 1) // tile_v
    return pl.pallas_call(
        _mm_body,
        grid=(grid,),
        in_specs=[
            pl.BlockSpec((batch, embed), lambda i: (0, 0)),
            pl.BlockSpec((embed, tile_v), lambda i: (0, i)),
            pl.BlockSpec((tile_v,), lambda i: (i,)),
        ],
        out_specs=pl.BlockSpec((tile_v, batch), lambda i: (i, 0)),
        out_shape=jax.ShapeDtypeStruct((vocab, batch), jnp.float32),
        compiler_params=pltpu.CompilerParams(
            dimension_semantics=("parallel",),
        ),
    )(means, Wt, b)


def kernel(context, emb_table, W, b):
    batch, ctx_len = context.shape
    vocab, embed = emb_table.shape
    ctx_flat = context.reshape(-1).astype(jnp.int32)
    table_pad = _transpose_pad(emb_table.T, vocab, embed)
    means = _sc_gather_mean(ctx_flat, table_pad, batch, ctx_len, embed)
    out_t = _tc_matmul_t(means, W.T, b, batch, embed, vocab)
    return out_t.T


# speedup vs baseline: 3.2019x; 1.0533x over previous
import functools

import jax
import jax.numpy as jnp
from jax import lax
from jax.experimental import pallas as pl
from jax.experimental.pallas import tpu as pltpu
from jax.experimental.pallas import tpu_sc as plsc

_NUM_CORES = 2
_NUM_SUBCORES = 16
_NUM_WORKERS = _NUM_CORES * _NUM_SUBCORES
_LANES = 16
_ROW = 128


def _t_body(embed, in_ref, out_ref):
    out_ref[:, :embed] = in_ref[...].T


def _transpose_pad(emb_t, vocab, embed):
    tile_v = 16384
    grid = (vocab + tile_v - 1) // tile_v
    return pl.pallas_call(
        functools.partial(_t_body, embed),
        grid=(grid,),
        in_specs=[pl.BlockSpec((embed, tile_v), lambda i: (0, i))],
        out_specs=pl.BlockSpec((tile_v, _ROW), lambda i: (i, 0)),
        out_shape=jax.ShapeDtypeStruct((vocab, _ROW), jnp.float32),
        compiler_params=pltpu.CompilerParams(
            dimension_semantics=("parallel",),
        ),
    )(emb_t)


def _sc_gather_mean(ctx_flat, table_pad, batch, ctx_len, embed):
    rows_per_worker = batch // _NUM_WORKERS
    idx_per_worker = rows_per_worker * ctx_len
    rows_per_chunk = 4
    idx_per_chunk = rows_per_chunk * ctx_len
    num_chunks = rows_per_worker // rows_per_chunk

    mesh = plsc.VectorSubcoreMesh(core_axis_name="c", subcore_axis_name="s")

    @functools.partial(
        pl.kernel,
        out_type=jax.ShapeDtypeStruct((batch, embed), jnp.float32),
        mesh=mesh,
        scratch_types=[
            pltpu.VMEM((idx_per_worker,), jnp.int32),
            pltpu.VMEM((idx_per_chunk, _ROW), jnp.float32),
            pltpu.VMEM((rows_per_worker, embed), jnp.float32),
        ],
    )
    def gather_mean(table_hbm, idx_hbm, out_hbm, idx_v, rows_v, acc_v):
        wid = lax.axis_index("s") * _NUM_CORES + lax.axis_index("c")
        idx_base = wid * idx_per_worker
        row_base = wid * rows_per_worker
        pltpu.sync_copy(idx_hbm.at[pl.ds(idx_base, idx_per_worker)], idx_v)

        scale = jnp.float32(1.0 / ctx_len)

        @pl.loop(0, num_chunks)
        def _(c):
            pltpu.sync_copy(
                table_hbm.at[idx_v.at[pl.ds(c * idx_per_chunk, idx_per_chunk)]],
                rows_v,
            )
            for r in range(rows_per_chunk):
                for q in range(embed // _LANES):
                    sl = pl.ds(q * _LANES, _LANES)
                    acc = rows_v[r * ctx_len, sl]
                    for j in range(1, ctx_len):
                        acc = acc + rows_v[r * ctx_len + j, sl]
                    acc_v[c * rows_per_chunk + r, sl] = acc * scale

        pltpu.sync_copy(acc_v, out_hbm.at[pl.ds(row_base, rows_per_worker)])

    return gather_mean(table_pad, ctx_flat)


def _mm_body(means_ref, wt_ref, b_ref, out_ref):
    m = means_ref[...].astype(jnp.bfloat16)
    wt = wt_ref[...].astype(jnp.bfloat16)
    acc = lax.dot_general(
        wt, m, (((0,), (1,)), ((), ())), preferred_element_type=jnp.float32
    )
    out_ref[...] = acc + b_ref[...][:, None]


def _tc_matmul_t(means, Wt, b, batch, embed, vocab):
    tile_v = 4096
    grid = (vocab + tile_v - 1) // tile_v
    return pl.pallas_call(
        _mm_body,
        grid=(grid,),
        in_specs=[
            pl.BlockSpec((batch, embed), lambda i: (0, 0)),
            pl.BlockSpec((embed, tile_v), lambda i: (0, i)),
            pl.BlockSpec((tile_v,), lambda i: (i,)),
        ],
        out_specs=pl.BlockSpec((tile_v, batch), lambda i: (i, 0)),
        out_shape=jax.ShapeDtypeStruct((vocab, batch), jnp.float32),
        compiler_params=pltpu.CompilerParams(
            dimension_semantics=("parallel",),
        ),
    )(means, Wt, b)


def kernel(context, emb_table, W, b):
    batch, ctx_len = context.shape
    vocab, embed = emb_table.shape
    ctx_flat = context.reshape(-1).astype(jnp.int32)
    table_pad = _transpose_pad(emb_table.T, vocab, embed)
    means = _sc_gather_mean(ctx_flat, table_pad, batch, ctx_len, embed)
    out_t = _tc_matmul_t(means, W.T, b, batch, embed, vocab)
    return out_t.T


# speedup vs baseline: 3.2778x; 1.0237x over previous
import functools

import jax
import jax.numpy as jnp
from jax import lax
from jax.experimental import pallas as pl
from jax.experimental.pallas import tpu as pltpu
from jax.experimental.pallas import tpu_sc as plsc

_NUM_CORES = 2
_NUM_SUBCORES = 16
_NUM_WORKERS = _NUM_CORES * _NUM_SUBCORES
_LANES = 16
_ROW = 128


def _t_body(embed, in_ref, out_ref):
    out_ref[:, :embed] = in_ref[...].T


def _transpose_pad(emb_t, vocab, embed):
    tile_v = 16384
    grid = (vocab + tile_v - 1) // tile_v
    return pl.pallas_call(
        functools.partial(_t_body, embed),
        grid=(grid,),
        in_specs=[pl.BlockSpec((embed, tile_v), lambda i: (0, i))],
        out_specs=pl.BlockSpec((tile_v, _ROW), lambda i: (i, 0)),
        out_shape=jax.ShapeDtypeStruct((vocab, _ROW), jnp.float32),
        compiler_params=pltpu.CompilerParams(
            dimension_semantics=("parallel",),
        ),
    )(emb_t)


def _sc_gather_mean(ctx_flat, table_pad, batch, ctx_len, embed):
    rows_per_worker = batch // _NUM_WORKERS
    idx_per_worker = rows_per_worker * ctx_len
    rows_per_chunk = 4
    idx_per_chunk = rows_per_chunk * ctx_len
    num_chunks = rows_per_worker // rows_per_chunk

    mesh = plsc.VectorSubcoreMesh(core_axis_name="c", subcore_axis_name="s")

    @functools.partial(
        pl.kernel,
        out_type=jax.ShapeDtypeStruct((batch, embed), jnp.float32),
        mesh=mesh,
        scratch_types=[
            pltpu.VMEM((idx_per_worker,), jnp.int32),
            pltpu.VMEM((idx_per_chunk, _ROW), jnp.float32),
            pltpu.VMEM((idx_per_chunk, _ROW), jnp.float32),
            pltpu.VMEM((rows_per_worker, embed), jnp.float32),
            pltpu.SemaphoreType.DMA,
            pltpu.SemaphoreType.DMA,
        ],
    )
    def gather_mean(table_hbm, idx_hbm, out_hbm, idx_v, rows0_v, rows1_v,
                    acc_v, sem0, sem1):
        wid = lax.axis_index("s") * _NUM_CORES + lax.axis_index("c")
        idx_base = wid * idx_per_worker
        row_base = wid * rows_per_worker
        pltpu.sync_copy(idx_hbm.at[pl.ds(idx_base, idx_per_worker)], idx_v)

        scale = jnp.float32(1.0 / ctx_len)

        def start(c, buf, sem):
            pltpu.async_copy(
                table_hbm.at[idx_v.at[pl.ds(c * idx_per_chunk, idx_per_chunk)]],
                buf, sem,
            )

        def wait(buf, sem):
            pltpu.make_async_copy(table_hbm.at[pl.ds(0, idx_per_chunk)],
                                  buf, sem).wait()

        def reduce_chunk(c, buf):
            for r in range(rows_per_chunk):
                for q in range(embed // _LANES):
                    sl = pl.ds(q * _LANES, _LANES)
                    acc = buf[r * ctx_len, sl]
                    for j in range(1, ctx_len):
                        acc = acc + buf[r * ctx_len + j, sl]
                    acc_v[c * rows_per_chunk + r, sl] = acc * scale

        start(0, rows0_v, sem0)
        start(1, rows1_v, sem1)

        @pl.loop(0, num_chunks // 2)
        def _(p):
            c = p * 2
            wait(rows0_v, sem0)
            reduce_chunk(c, rows0_v)

            @pl.when(p < num_chunks // 2 - 1)
            def _():
                start(c + 2, rows0_v, sem0)

            wait(rows1_v, sem1)
            reduce_chunk(c + 1, rows1_v)

            @pl.when(p < num_chunks // 2 - 1)
            def _():
                start(c + 3, rows1_v, sem1)

        pltpu.sync_copy(acc_v, out_hbm.at[pl.ds(row_base, rows_per_worker)])

    return gather_mean(table_pad, ctx_flat)


def _mm_body(means_ref, wt_ref, b_ref, out_ref):
    m = means_ref[...].astype(jnp.bfloat16)
    wt = wt_ref[...].astype(jnp.bfloat16)
    acc = lax.dot_general(
        wt, m, (((0,), (1,)), ((), ())), preferred_element_type=jnp.float32
    )
    out_ref[...] = acc + b_ref[...][:, None]


def _tc_matmul_t(means, Wt, b, batch, embed, vocab):
    tile_v = 4096
    grid = (vocab + tile_v - 1) // tile_v
    return pl.pallas_call(
        _mm_body,
        grid=(grid,),
        in_specs=[
            pl.BlockSpec((batch, embed), lambda i: (0, 0)),
            pl.BlockSpec((embed, tile_v), lambda i: (0, i)),
            pl.BlockSpec((tile_v,), lambda i: (i,)),
        ],
        out_specs=pl.BlockSpec((tile_v, batch), lambda i: (i, 0)),
        out_shape=jax.ShapeDtypeStruct((vocab, batch), jnp.float32),
        compiler_params=pltpu.CompilerParams(
            dimension_semantics=("parallel",),
        ),
    )(means, Wt, b)


def kernel(context, emb_table, W, b):
    batch, ctx_len = context.shape
    vocab, embed = emb_table.shape
    ctx_flat = context.reshape(-1).astype(jnp.int32)
    table_pad = _transpose_pad(emb_table.T, vocab, embed)
    means = _sc_gather_mean(ctx_flat, table_pad, batch, ctx_len, embed)
    out_t = _tc_matmul_t(means, W.T, b, batch, embed, vocab)
    return out_t.T
